# Initial kernel scaffold; baseline (speedup 1.0000x reference)
import jax
import jax.numpy as jnp
from jax.experimental import pallas as pl


def kernel(seq):
    raise NotImplementedError("write your pallas kernel here")


# speedup vs baseline: 1.0659x; 1.0659x over previous
import functools

import numpy as np
import jax
import jax.numpy as jnp
from jax import lax
from jax.experimental import pallas as pl

_N_ROWS = 16384
_SEQ_LEN = 200
_NA = 25
_W = _SEQ_LEN * _NA

_KS0 = np.uint32(0)
_KS1 = np.uint32(42)
_KS2 = np.uint32(0x1BD11BDA ^ 42)
_ROTS = ((13, 15, 26, 6), (17, 29, 16, 24))
_INJECT = (
    (_KS1, np.uint32(_KS2 + np.uint32(1))),
    (_KS2, np.uint32(_KS0 + np.uint32(2))),
    (_KS0, np.uint32(_KS1 + np.uint32(3))),
    (_KS1, np.uint32(_KS2 + np.uint32(4))),
    (_KS2, np.uint32(_KS0 + np.uint32(5))),
)

_JDX = np.arange(_W) // _NA
_REP_NP = (_JDX[None, :] == np.arange(_SEQ_LEN)[:, None]).astype(np.float32)
_REP_BF16 = jnp.asarray(_REP_NP, dtype=jnp.bfloat16)
_SUM_BF16 = jnp.asarray(_REP_NP.T, dtype=jnp.bfloat16)


def _threefry_bits(g):
  x1 = g + _KS1
  x0 = x1
  first = True
  for grp in range(5):
    for r in _ROTS[grp % 2]:
      if first:
        first = False
      else:
        x0 = x0 + x1
      x1 = ((x1 << np.uint32(r)) | (x1 >> np.uint32(32 - r))) ^ x0
    a, b = _INJECT[grp]
    x0 = x0 + a
    x1 = x1 + b
  return x0 ^ x1


def _dense_kernel(seq_ref, rep_ref, sum_ref, out_ref, *, block_rows):
  pid = pl.program_id(0)
  qi = lax.broadcasted_iota(jnp.int32, (block_rows, _W), 1)
  ri = lax.broadcasted_iota(jnp.int32, (block_rows, _W), 0)

  base = (pid * block_rows * _W).astype(jnp.uint32)
  g = base + ri.astype(jnp.uint32) * np.uint32(_W) + qi.astype(jnp.uint32)
  bits = _threefry_bits(g)
  u = lax.bitcast_convert_type(
      (bits >> np.uint32(9)) | np.uint32(0x3F800000), jnp.float32) - 1.0

  seq_bf = seq_ref[...].astype(jnp.float32).astype(jnp.bfloat16)
  sval = jnp.dot(seq_bf, rep_ref[...], preferred_element_type=jnp.float32)

  rowsum = jnp.dot(u.astype(jnp.bfloat16), sum_ref[...],
                   preferred_element_type=jnp.float32)
  denom = jnp.dot(rowsum.astype(jnp.bfloat16), rep_ref[...],
                  preferred_element_type=jnp.float32)

  jdx = (qi * 10486) >> 18
  kdx = (qi - _NA * jdx).astype(jnp.float32)

  onehot = jnp.where(kdx == sval, 1.0, 0.0)
  out_ref[...] = jnp.where(sval == 24.0, u / denom, onehot)


@jax.jit
def kernel(seq):
  block_rows = 64
  out = pl.pallas_call(
      functools.partial(_dense_kernel, block_rows=block_rows),
      grid=(_N_ROWS // block_rows,),
      in_specs=[
          pl.BlockSpec((block_rows, _SEQ_LEN), lambda i: (i, 0)),
          pl.BlockSpec((_SEQ_LEN, _W), lambda i: (0, 0)),
          pl.BlockSpec((_W, _SEQ_LEN), lambda i: (0, 0)),
      ],
      out_specs=pl.BlockSpec((block_rows, _W), lambda i: (i, 0)),
      out_shape=jax.ShapeDtypeStruct((_N_ROWS, _W), jnp.float32),
  )(seq, _REP_BF16, _SUM_BF16)
  return out.reshape(_N_ROWS, _SEQ_LEN, _NA)
